# Initial kernel scaffold; baseline (speedup 1.0000x reference)
import jax
import jax.numpy as jnp
from jax.experimental import pallas as pl


def kernel(x, edge_index, W_gat, att_src, att_dst, b_gat, W_sage_l, b_sage_l, W_sage_r, W_proj, b_proj, gamma, beta):
    raise NotImplementedError("write your pallas kernel here")


# speedup vs baseline: 27.0516x; 27.0516x over previous
import functools

import jax
import jax.numpy as jnp
from jax import lax
from jax.experimental import pallas as pl
from jax.experimental.pallas import tpu as pltpu
from jax.experimental.pallas import tpu_sc as plsc

N = 10000
E = 320000
DIM = 128
H = 4
DH = DIM // H

NP = 10240
NS = 16
C = 80
EPT = E // NS
NCHUNK = EPT // C
RPT = NP // NS

BLK = 400
GRID = N // BLK


def _tc_pre(x_ref, wg_ref, amat_ref, hx_ref, atab_ref):
    blk = x_ref[...]
    hval = jnp.dot(blk, wg_ref[...], preferred_element_type=jnp.float32)
    outv = jnp.where(pl.program_id(0) < GRID, hval, blk)
    hx_ref[...] = outv
    atab_ref[...] = jnp.dot(outv, amat_ref[...], preferred_element_type=jnp.float32)


def _sc_edge(hx_hbm, atab_hbm, src_hbm, dst_hbm,
             outg, outx, outw,
             src_v, dst_v, src2_v, rows_v, ts_v, td_v, w8_v,
             acc_sh, acc8_sh, sem):
    cid = lax.axis_index("c")
    sid = lax.axis_index("s")
    iota = lax.iota(jnp.int32, 16)
    z16 = jnp.zeros((16,), jnp.float32)
    zi16 = jnp.zeros((16,), jnp.int32)

    def _zrow(i, carry):
        for k in range(8):
            rows_v[i, pl.ds(k * 16, 16)] = z16
        return carry
    lax.fori_loop(0, C, _zrow, 0)

    def _zw8(i, carry):
        w8_v[i, pl.ds(0, 16)] = z16
        ts_v[i, pl.ds(0, 16)] = z16
        td_v[i, pl.ds(0, 16)] = z16
        return carry
    lax.fori_loop(0, C, _zw8, 0)

    for j in range(RPT // C):
        pltpu.sync_copy(rows_v, acc_sh.at[pl.ds(sid * RPT + j * C, C)])
        pltpu.sync_copy(w8_v, acc8_sh.at[pl.ds(sid * RPT + j * C, C)])

    plsc.subcore_barrier()

    offv = jnp.full((16,), 0, jnp.int32) + cid * N

    def chunk(ci, carry):
        base = sid * EPT + ci * C
        pltpu.sync_copy(src_hbm.at[pl.ds(base, C)], src_v)
        pltpu.sync_copy(dst_hbm.at[pl.ds(base, C)], dst_v)
        for g in range(C // 16):
            sl = pl.ds(g * 16, 16)
            src2_v[sl] = src_v[sl] + offv
        pltpu.async_copy(hx_hbm.at[src2_v], rows_v, sem).wait()

        @pl.when(cid == 0)
        def _():
            pltpu.async_copy(atab_hbm.at[src_v], ts_v, sem).wait()
            pltpu.async_copy(atab_hbm.at[dst_v], td_v, sem).wait()

        def scale(ei, c2):
            e16 = (ts_v[ei, pl.ds(0, 16)]
                   + lax.rev(td_v[ei, pl.ds(0, 16)], dimensions=(0,)))
            w16 = jnp.exp(jnp.maximum(e16, e16 * 0.2))
            w8row = jnp.where(iota < 4, w16,
                              jnp.where(iota == 4, 1.0, 0.0).astype(jnp.float32))
            w8_v[ei, pl.ds(0, 16)] = w8row
            for hh in range(H):
                wv = jnp.full((16,), w16[hh])
                for k in range(2):
                    sl = pl.ds(hh * 32 + k * 16, 16)
                    rows_v[ei, sl] = rows_v[ei, sl] * wv
            return c2
        lax.fori_loop(0, C, scale, 0)

        pltpu.sync_copy(rows_v, acc_sh.at[dst_v], add=True)
        pltpu.sync_copy(w8_v, acc8_sh.at[dst_v], add=True)
        return carry
    lax.fori_loop(0, NCHUNK, chunk, 0)

    plsc.subcore_barrier()

    rb = pl.ds(sid * RPT, RPT)
    @pl.when(cid == 0)
    def _():
        pltpu.sync_copy(acc_sh.at[rb], outg.at[rb])
        pltpu.sync_copy(acc8_sh.at[rb], outw.at[rb])

    @pl.when(cid != 0)
    def _():
        pltpu.sync_copy(acc_sh.at[rb], outx.at[rb])


def _tc_post(accg_ref, accx_ref, wcnt_ref, h_ref, atab_ref, x_ref,
             p164_ref, q84_ref, q81_ref, r_ref,
             wsl_ref, wsr_ref, wp1_ref, wp2_ref,
             bgat_ref, bsl_ref, bproj_ref, gamma_ref, beta_ref, out_ref):
    f32 = jnp.float32
    atab = atab_ref[...]
    esum4 = jnp.dot(atab, p164_ref[...], preferred_element_type=f32)
    wself4 = jnp.exp(jnp.maximum(esum4, esum4 * 0.2))
    den4 = jnp.dot(wcnt_ref[...], q84_ref[...], preferred_element_type=f32) + wself4 + 1e-16
    cnt1 = jnp.dot(wcnt_ref[...], q81_ref[...], preferred_element_type=f32)
    wrep = jnp.dot(wself4, r_ref[...], preferred_element_type=f32)
    denrep = jnp.dot(den4, r_ref[...], preferred_element_type=f32)
    x = x_ref[...]
    h = h_ref[...]
    gat = (accg_ref[...] + h * wrep) / denrep + bgat_ref[...]
    mean = accx_ref[...] / jnp.maximum(cnt1, 1.0)
    sage = (jnp.dot(mean, wsl_ref[...], preferred_element_type=f32) + bsl_ref[...]
            + jnp.dot(x, wsr_ref[...], preferred_element_type=f32))
    o = (jnp.dot(gat, wp1_ref[...], preferred_element_type=f32)
         + jnp.dot(sage, wp2_ref[...], preferred_element_type=f32) + bproj_ref[...])
    y = o + x
    mu = jnp.mean(y, axis=-1, keepdims=True)
    d = y - mu
    var = jnp.mean(d * d, axis=-1, keepdims=True)
    out_ref[...] = gamma_ref[...] * d * lax.rsqrt(var + 1e-5) + beta_ref[...]


def kernel(x, edge_index, W_gat, att_src, att_dst, b_gat, W_sage_l, b_sage_l,
           W_sage_r, W_proj, b_proj, gamma, beta):
    f32 = jnp.float32
    src = edge_index[0]
    dst = edge_index[1]

    amat = jnp.zeros((DIM, 16), f32)
    for hh in range(H):
        amat = amat.at[hh * DH:(hh + 1) * DH, hh].set(att_src[hh])
        amat = amat.at[hh * DH:(hh + 1) * DH, 15 - hh].set(att_dst[hh])

    hx, atab = pl.pallas_call(
        _tc_pre,
        grid=(2 * GRID,),
        in_specs=[
            pl.BlockSpec((BLK, DIM), lambda i: (i % GRID, 0)),
            pl.BlockSpec((DIM, DIM), lambda i: (0, 0)),
            pl.BlockSpec((DIM, 16), lambda i: (0, 0)),
        ],
        out_specs=[
            pl.BlockSpec((BLK, DIM), lambda i: (i, 0)),
            pl.BlockSpec((BLK, 16), lambda i: (i, 0)),
        ],
        out_shape=[
            jax.ShapeDtypeStruct((2 * N, DIM), f32),
            jax.ShapeDtypeStruct((2 * N, 16), f32),
        ],
    )(x, W_gat, amat)

    sc_edge = functools.partial(
        pl.kernel,
        mesh=plsc.VectorSubcoreMesh(core_axis_name="c", subcore_axis_name="s"),
        compiler_params=pltpu.CompilerParams(use_tc_tiling_on_sc=False),
        out_type=[
            jax.ShapeDtypeStruct((NP, DIM), f32),
            jax.ShapeDtypeStruct((NP, DIM), f32),
            jax.ShapeDtypeStruct((NP, 16), f32),
        ],
        scratch_types=[
            pltpu.VMEM((C,), jnp.int32),
            pltpu.VMEM((C,), jnp.int32),
            pltpu.VMEM((C,), jnp.int32),
            pltpu.VMEM((C, DIM), f32),
            pltpu.VMEM((C, 16), f32),
            pltpu.VMEM((C, 16), f32),
            pltpu.VMEM((C, 16), f32),
            pltpu.VMEM_SHARED((NP, DIM), f32),
            pltpu.VMEM_SHARED((NP, 16), f32),
            pltpu.SemaphoreType.DMA,
        ],
    )(_sc_edge)
    accg, accx, wcnt = sc_edge(hx, atab, src, dst)

    p164 = jnp.zeros((16, H), f32)
    for hh in range(H):
        p164 = p164.at[hh, hh].set(1.0).at[15 - hh, hh].set(1.0)
    q84 = jnp.zeros((16, H), f32).at[:4, :4].set(jnp.eye(H, dtype=f32))
    q81 = jnp.zeros((16, 1), f32).at[4, 0].set(1.0)
    rmat = jnp.repeat(jnp.eye(H, dtype=f32), DH, axis=1)

    row = lambda v: v.reshape(1, DIM)
    out = pl.pallas_call(
        _tc_post,
        grid=(GRID,),
        in_specs=[
            pl.BlockSpec((BLK, DIM), lambda i: (i, 0)),
            pl.BlockSpec((BLK, DIM), lambda i: (i, 0)),
            pl.BlockSpec((BLK, 16), lambda i: (i, 0)),
            pl.BlockSpec((BLK, DIM), lambda i: (i, 0)),
            pl.BlockSpec((BLK, 16), lambda i: (i, 0)),
            pl.BlockSpec((BLK, DIM), lambda i: (i, 0)),
            pl.BlockSpec((16, H), lambda i: (0, 0)),
            pl.BlockSpec((16, H), lambda i: (0, 0)),
            pl.BlockSpec((16, 1), lambda i: (0, 0)),
            pl.BlockSpec((H, DIM), lambda i: (0, 0)),
            pl.BlockSpec((DIM, DIM), lambda i: (0, 0)),
            pl.BlockSpec((DIM, DIM), lambda i: (0, 0)),
            pl.BlockSpec((DIM, DIM), lambda i: (0, 0)),
            pl.BlockSpec((DIM, DIM), lambda i: (0, 0)),
            pl.BlockSpec((1, DIM), lambda i: (0, 0)),
            pl.BlockSpec((1, DIM), lambda i: (0, 0)),
            pl.BlockSpec((1, DIM), lambda i: (0, 0)),
            pl.BlockSpec((1, DIM), lambda i: (0, 0)),
            pl.BlockSpec((1, DIM), lambda i: (0, 0)),
        ],
        out_specs=pl.BlockSpec((BLK, DIM), lambda i: (i, 0)),
        out_shape=jax.ShapeDtypeStruct((N, DIM), f32),
    )(accg, accx, wcnt, hx, atab, x,
      p164, q84, q81, rmat,
      W_sage_l, W_sage_r, W_proj[:DIM], W_proj[DIM:],
      row(b_gat), row(b_sage_l), row(b_proj), row(gamma), row(beta))
    return out
